# Initial kernel scaffold; baseline (speedup 1.0000x reference)
import functools

import jax
import jax.numpy as jnp
import numpy as np
from jax.experimental import pallas as pl

NUM_POS = 4096
NUM_NEG = 999
TOP_N = 10
SIZE = NUM_POS * (NUM_NEG + 1)
ROW_PAD = 1024


@functools.cache
def _row_masks() -> np.ndarray:
    keys = jax.random.split(jax.random.key(42), NUM_POS)
    perms = jax.jit(jax.vmap(lambda k: jax.random.permutation(k, NUM_NEG + 1)))(keys)
    perms = np.asarray(perms)
    inv = np.argsort(perms, axis=1)
    v = perms[:, -1]
    p = inv[:, NUM_NEG]
    m = np.ones((NUM_POS, NUM_NEG), dtype=np.float32)
    m[inv[:, :NUM_NEG] < p[:, None]] = 2.0
    full = v == NUM_NEG
    m[full, :] = 2.0
    rows = np.where(~full)[0]
    m[rows, v[~full]] = 0.0
    padded = np.zeros((NUM_POS, ROW_PAD), dtype=np.float32)
    padded[:, :NUM_NEG] = m
    return padded


ROWS_PER_BLK = 128
GRID = NUM_POS // ROWS_PER_BLK


def _metric_body(pv_ref, m_ref, hit_ref, ndcg_ref):
    pv = pv_ref[...]
    m = m_ref[...]
    pos = pv[:, NUM_NEG:NUM_NEG + 1]
    gt = jnp.logical_and(pv > pos, m > 0.5)
    ge = jnp.logical_and(pv == pos, m > 1.5)
    cnt = jnp.sum(gt.astype(jnp.float32) + ge.astype(jnp.float32), axis=1)
    rank = cnt + 1.0
    hit = (rank <= TOP_N).astype(jnp.float32)
    ndcg = hit / jnp.log2(rank + 1.0)
    hit_ref[...] = hit[None, :]
    ndcg_ref[...] = ndcg[None, :]


def _metrics(pv_pad):
    m = jnp.asarray(_row_masks())
    hit, ndcg = pl.pallas_call(
        _metric_body,
        grid=(GRID,),
        in_specs=[
            pl.BlockSpec((ROWS_PER_BLK, ROW_PAD), lambda i: (i, 0)),
            pl.BlockSpec((ROWS_PER_BLK, ROW_PAD), lambda i: (i, 0)),
        ],
        out_specs=[
            pl.BlockSpec((1, ROWS_PER_BLK), lambda i: (i, 0)),
            pl.BlockSpec((1, ROWS_PER_BLK), lambda i: (i, 0)),
        ],
        out_shape=[
            jax.ShapeDtypeStruct((GRID, ROWS_PER_BLK), jnp.float32),
            jax.ShapeDtypeStruct((GRID, ROWS_PER_BLK), jnp.float32),
        ],
    )(pv_pad, m)
    return hit.reshape(NUM_POS), ndcg.reshape(NUM_POS)


def kernel(n, num, predict_val, num_pos, index):
    order = jnp.argsort(index)
    pv = predict_val[order]
    neg = pv[NUM_POS:].reshape(NUM_POS, NUM_NEG)
    pos = pv[:NUM_POS]
    pv_pad = jnp.concatenate(
        [neg, pos[:, None], jnp.zeros((NUM_POS, ROW_PAD - NUM_NEG - 1), jnp.float32)],
        axis=1,
    )
    hits, ndcgs = _metrics(pv_pad)
    Hits = jnp.sum(hits) / num_pos
    ndcg = jnp.sum(ndcgs) / num_pos
    return Hits, ndcg, hits, ndcgs


# speedup vs baseline: 1.4131x; 1.4131x over previous
import functools

import jax
import jax.numpy as jnp
import numpy as np
from jax.experimental import pallas as pl

NUM_POS = 4096
NUM_NEG = 999
TOP_N = 10
SIZE = NUM_POS * (NUM_NEG + 1)
ROW_PAD = 1024


@functools.cache
def _row_masks() -> np.ndarray:
    keys = jax.random.split(jax.random.key(42), NUM_POS)
    perms = jax.jit(jax.vmap(lambda k: jax.random.permutation(k, NUM_NEG + 1)))(keys)
    perms = np.asarray(perms)
    inv = np.argsort(perms, axis=1)
    v = perms[:, -1]
    p = inv[:, NUM_NEG]
    m = np.ones((NUM_POS, NUM_NEG), dtype=np.float32)
    m[inv[:, :NUM_NEG] < p[:, None]] = 2.0
    full = v == NUM_NEG
    m[full, :] = 2.0
    rows = np.where(~full)[0]
    m[rows, v[~full]] = 0.0
    padded = np.zeros((NUM_POS, ROW_PAD), dtype=np.float32)
    padded[:, :NUM_NEG] = m
    return padded


_MASKS = _row_masks()

ROWS_PER_BLK = 128
GRID = NUM_POS // ROWS_PER_BLK


def _metric_body(pv_ref, m_ref, hit_ref, ndcg_ref):
    pv = pv_ref[...]
    m = m_ref[...]
    pos = pv[:, NUM_NEG:NUM_NEG + 1]
    gt = jnp.logical_and(pv > pos, m > 0.5)
    ge = jnp.logical_and(pv == pos, m > 1.5)
    cnt = jnp.sum(gt.astype(jnp.float32) + ge.astype(jnp.float32), axis=1)
    rank = cnt + 1.0
    hit = (rank <= TOP_N).astype(jnp.float32)
    ndcg = hit / jnp.log2(rank + 1.0)
    hit_ref[...] = hit[None, None, :]
    ndcg_ref[...] = ndcg[None, None, :]


def _metrics(pv_pad):
    m = jnp.asarray(_MASKS)
    hit, ndcg = pl.pallas_call(
        _metric_body,
        grid=(GRID,),
        in_specs=[
            pl.BlockSpec((ROWS_PER_BLK, ROW_PAD), lambda i: (i, 0)),
            pl.BlockSpec((ROWS_PER_BLK, ROW_PAD), lambda i: (i, 0)),
        ],
        out_specs=[
            pl.BlockSpec((1, 1, ROWS_PER_BLK), lambda i: (i, 0, 0)),
            pl.BlockSpec((1, 1, ROWS_PER_BLK), lambda i: (i, 0, 0)),
        ],
        out_shape=[
            jax.ShapeDtypeStruct((GRID, 1, ROWS_PER_BLK), jnp.float32),
            jax.ShapeDtypeStruct((GRID, 1, ROWS_PER_BLK), jnp.float32),
        ],
    )(pv_pad, m)
    return hit.reshape(NUM_POS), ndcg.reshape(NUM_POS)


def kernel(n, num, predict_val, num_pos, index):
    order = jnp.argsort(index)
    pv = predict_val[order]
    neg = pv[NUM_POS:].reshape(NUM_POS, NUM_NEG)
    pos = pv[:NUM_POS]
    pv_pad = jnp.concatenate(
        [neg, pos[:, None], jnp.zeros((NUM_POS, ROW_PAD - NUM_NEG - 1), jnp.float32)],
        axis=1,
    )
    hits, ndcgs = _metrics(pv_pad)
    Hits = jnp.sum(hits) / num_pos
    ndcg = jnp.sum(ndcgs) / num_pos
    return Hits, ndcg, hits, ndcgs
